# baseline (speedup 1.0000x reference)
import functools

import jax
import jax.numpy as jnp
from jax import lax
from jax.experimental import pallas as pl
from jax.experimental.pallas import tpu as pltpu
from jax.experimental.pallas import tpu_sc as plsc

F32 = jnp.float32
_NSUB = 16
_K = 128
_B = 2000


def _row_split(N):
    r0 = -(-((N + _NSUB - 1) // _NSUB) // 8) * 8
    last = N - (_NSUB - 1) * r0
    assert last > 0 and last % 8 == 0 and N % 8 == 0
    return r0, last


BF16 = jnp.bfloat16


@functools.cache
def _sc_eadeg(N, E):
    nw = 2 * _NSUB
    eps = E // nw
    nfull = eps // _K
    tail = eps % _K
    npairs = nfull // 2
    odd = nfull - 2 * npairs
    r0, rlast = _row_split(N)
    mesh = plsc.VectorSubcoreMesh(core_axis_name="c", subcore_axis_name="s")

    def body(ea_hbm, dst_hbm, z_hbm, out_hbm, acc, vA, vB, dA, dB, dtail,
             semA, semB):
        c = lax.axis_index("c")
        s = lax.axis_index("s")

        @pl.when(s < _NSUB - 1)
        def _():
            pltpu.sync_copy(z_hbm, acc.at[pl.ds(s * r0, r0)])

        @pl.when(s == _NSUB - 1)
        def _():
            pltpu.sync_copy(z_hbm.at[pl.ds(0, rlast)],
                            acc.at[pl.ds((_NSUB - 1) * r0, rlast)])

        plsc.subcore_barrier()
        base = (s * 2 + c) * eps

        def load(j, v, dref, sem):
            off = base + j * _K
            pltpu.async_copy(ea_hbm.at[pl.ds(off, _K)], v, sem)
            pltpu.async_copy(dst_hbm.at[pl.ds(off, _K)], dref, sem)

        def wait(j, v, dref, sem):
            off = base + j * _K
            pltpu.make_async_copy(ea_hbm.at[pl.ds(off, _K)], v, sem).wait()
            pltpu.make_async_copy(dst_hbm.at[pl.ds(off, _K)], dref, sem).wait()

        def scatter(v, dref):
            pltpu.sync_copy(v, acc.at[dref], add=True)

        load(0, vA, dA, semA)

        def pair(i, carry):
            j0 = 2 * i
            load(j0 + 1, vB, dB, semB)
            wait(j0, vA, dA, semA)
            scatter(vA, dA)

            @pl.when(j0 + 2 < nfull)
            def _():
                load(j0 + 2, vA, dA, semA)

            wait(j0 + 1, vB, dB, semB)
            scatter(vB, dB)
            return carry

        lax.fori_loop(0, npairs, pair, 0)
        if odd:
            j = nfull - 1
            wait(j, vA, dA, semA)
            scatter(vA, dA)
        if tail:
            off = base + nfull * _K
            pltpu.sync_copy(ea_hbm.at[pl.ds(off, tail)], vA.at[pl.ds(0, tail)])
            pltpu.sync_copy(dst_hbm.at[pl.ds(off, tail)], dtail)
            pltpu.sync_copy(vA.at[pl.ds(0, tail)], acc.at[dtail], add=True)
        plsc.subcore_barrier()

        @pl.when(s < _NSUB - 1)
        def _():
            pltpu.sync_copy(acc.at[pl.ds(s * r0, r0)],
                            out_hbm.at[c, pl.ds(s * r0, r0)])

        @pl.when(s == _NSUB - 1)
        def _():
            pltpu.sync_copy(acc.at[pl.ds((_NSUB - 1) * r0, rlast)],
                            out_hbm.at[c, pl.ds((_NSUB - 1) * r0, rlast)])

    return pl.kernel(
        body,
        out_type=jax.ShapeDtypeStruct((2, N, 32), BF16),
        mesh=mesh,
        compiler_params=pltpu.CompilerParams(use_tc_tiling_on_sc=False),
        scratch_types=[
            pltpu.VMEM_SHARED((N, 32), BF16),
            pltpu.VMEM((_K, 32), BF16),
            pltpu.VMEM((_K, 32), BF16),
            pltpu.VMEM((_K,), jnp.int32),
            pltpu.VMEM((_K,), jnp.int32),
            pltpu.VMEM((max(tail, 16),), jnp.int32),
            pltpu.SemaphoreType.DMA,
            pltpu.SemaphoreType.DMA,
        ],
    )


def _tc_ea32(ea):
    E, EF = ea.shape
    B2 = 8000

    def body(x_ref, o_ref):
        v = x_ref[...].astype(BF16)
        o_ref[...] = jnp.concatenate(
            [v, jnp.ones((B2, 32 - EF), BF16)], axis=1)

    return pl.pallas_call(
        body,
        grid=(E // B2,),
        in_specs=[pl.BlockSpec((B2, EF), lambda i: (i, 0))],
        out_specs=pl.BlockSpec((B2, 32), lambda i: (i, 0)),
        out_shape=jax.ShapeDtypeStruct((E, 32), BF16),
    )(ea)


BF16 = jnp.bfloat16


@functools.cache
def _sc_seg(N, E):
    eps = E // _NSUB
    nfull = eps // _K
    tail = eps % _K
    npairs = nfull // 2
    assert nfull == 2 * npairs
    r0, rlast = _row_split(N)
    mesh = plsc.VectorSubcoreMesh(core_axis_name="c", subcore_axis_name="s")

    def body(xh_hbm, src_hbm, dst_hbm, z_hbm, out_hbm, acc,
             gbufA, gbufB, sidxA, sidxB, gidxA, gidxB, didxA, didxB, dtail,
             semA, semB, semIA, semIB):
        c = lax.axis_index("c")
        s = lax.axis_index("s")
        base_row = c * N

        @pl.when(s < _NSUB - 1)
        def _():
            pltpu.sync_copy(z_hbm, acc.at[pl.ds(s * r0, r0)])

        @pl.when(s == _NSUB - 1)
        def _():
            pltpu.sync_copy(z_hbm.at[pl.ds(0, rlast)],
                            acc.at[pl.ds((_NSUB - 1) * r0, rlast)])

        plsc.subcore_barrier()
        base = s * eps

        def load_idx(j, sidx, didx, semI):
            off = base + j * _K
            pltpu.async_copy(src_hbm.at[pl.ds(off, _K)], sidx, semI)
            pltpu.async_copy(dst_hbm.at[pl.ds(off, _K)], didx, semI)

        def wait_idx(j, sidx, didx, semI):
            off = base + j * _K
            pltpu.make_async_copy(src_hbm.at[pl.ds(off, _K)], sidx, semI).wait()
            pltpu.make_async_copy(dst_hbm.at[pl.ds(off, _K)], didx, semI).wait()

        def compute_gidx(sidx, gidx, n16=_K // 16):
            for t in range(n16):
                sl = pl.ds(t * 16, 16)
                gidx[sl] = sidx[sl] + base_row

        def start_gather(gidx, gbuf, sem):
            pltpu.async_copy(xh_hbm.at[gidx], gbuf, sem)

        def wait_gather(gbuf, sem):
            pltpu.make_async_copy(xh_hbm.at[pl.ds(0, _K)], gbuf, sem).wait()

        def scatter(gbuf, didx):
            pltpu.sync_copy(gbuf, acc.at[didx], add=True)

        load_idx(0, sidxA, didxA, semIA)

        def pair(i, carry):
            j0 = 2 * i

            @pl.when(i > 0)
            def _():
                wait_gather(gbufB, semB)
                scatter(gbufB, didxB)

            load_idx(j0 + 1, sidxB, didxB, semIB)
            wait_idx(j0, sidxA, didxA, semIA)
            compute_gidx(sidxA, gidxA)
            start_gather(gidxA, gbufA, semA)
            wait_idx(j0 + 1, sidxB, didxB, semIB)
            compute_gidx(sidxB, gidxB)
            wait_gather(gbufA, semA)
            scatter(gbufA, didxA)
            start_gather(gidxB, gbufB, semB)

            @pl.when(i + 1 < npairs)
            def _():
                load_idx(j0 + 2, sidxA, didxA, semIA)

            return carry

        lax.fori_loop(0, npairs, pair, 0)
        wait_gather(gbufB, semB)
        scatter(gbufB, didxB)
        if tail:
            off = base + nfull * _K
            pltpu.sync_copy(src_hbm.at[pl.ds(off, tail)],
                            sidxA.at[pl.ds(0, tail)])
            pltpu.sync_copy(dst_hbm.at[pl.ds(off, tail)], dtail)
            compute_gidx(sidxA, gidxA, tail // 16)
            pltpu.async_copy(xh_hbm.at[gidxA.at[pl.ds(0, tail)]],
                             gbufA.at[pl.ds(0, tail)], semA).wait()
            pltpu.sync_copy(gbufA.at[pl.ds(0, tail)], acc.at[dtail], add=True)
        plsc.subcore_barrier()

        @pl.when(s < _NSUB - 1)
        def _():
            pltpu.sync_copy(acc.at[pl.ds(s * r0, r0)],
                            out_hbm.at[c, pl.ds(s * r0, r0)])

        @pl.when(s == _NSUB - 1)
        def _():
            pltpu.sync_copy(acc.at[pl.ds((_NSUB - 1) * r0, rlast)],
                            out_hbm.at[c, pl.ds((_NSUB - 1) * r0, rlast)])

    return pl.kernel(
        body,
        out_type=jax.ShapeDtypeStruct((2, N, 32), BF16),
        mesh=mesh,
        compiler_params=pltpu.CompilerParams(use_tc_tiling_on_sc=False),
        scratch_types=[
            pltpu.VMEM_SHARED((N, 32), BF16),
            pltpu.VMEM((_K, 32), BF16),
            pltpu.VMEM((_K, 32), BF16),
            pltpu.VMEM((_K,), jnp.int32),
            pltpu.VMEM((_K,), jnp.int32),
            pltpu.VMEM((_K,), jnp.int32),
            pltpu.VMEM((_K,), jnp.int32),
            pltpu.VMEM((_K,), jnp.int32),
            pltpu.VMEM((_K,), jnp.int32),
            pltpu.VMEM((max(tail, 16),), jnp.int32),
            pltpu.SemaphoreType.DMA,
            pltpu.SemaphoreType.DMA,
            pltpu.SemaphoreType.DMA,
            pltpu.SemaphoreType.DMA,
        ],
    )


def _tc_in(x, Wn, bn):
    N, Fin = x.shape

    def body(x_ref, w_ref, b_ref, o_ref):
        xp = jnp.dot(x_ref[...], w_ref[...], preferred_element_type=F32) \
            + b_ref[...]
        xb = xp.astype(BF16)
        o_ref[0] = xb[:, :32]
        o_ref[1] = xb[:, 32:]

    return pl.pallas_call(
        body,
        grid=(N // _B,),
        in_specs=[
            pl.BlockSpec((_B, Fin), lambda i: (i, 0)),
            pl.BlockSpec((Fin, 64), lambda i: (0, 0)),
            pl.BlockSpec((1, 64), lambda i: (0, 0)),
        ],
        out_specs=pl.BlockSpec((2, _B, 32), lambda i: (0, i, 0)),
        out_shape=jax.ShapeDtypeStruct((2, N, 32), BF16),
    )(x, Wn, bn.reshape(1, 64))


def _layer_z(xh_ref, s_ref, ed_ref, a_ref, bm_ref, c_ref, d_ref, cp_ref, bu_ref):
    xp = jnp.concatenate([xh_ref[0], xh_ref[1]], axis=1).astype(F32)
    sf = jnp.concatenate([s_ref[0], s_ref[1]], axis=1).astype(F32)
    edc = ed_ref[0].astype(F32) + ed_ref[1].astype(F32)
    ea = edc[:, :16]
    deg = edc[:, 16:17] + 1.0
    z = (jnp.dot(xp, a_ref[...], preferred_element_type=F32)
         + (jnp.dot(xp, bm_ref[...], preferred_element_type=F32)
            + cp_ref[...]) * deg
         + jnp.dot(ea, c_ref[...], preferred_element_type=F32)
         + jnp.dot(sf, d_ref[...], preferred_element_type=F32)
         + bu_ref[...])
    return jnp.maximum(z, 0.0)


def _tc_mid(xh, S, eadeg, A, Bm, C, D, cp, bu, Wn2, bn2):
    N = xh.shape[1]

    def body(xh_ref, s_ref, ed_ref, a_ref, bm_ref, c_ref, d_ref, cp_ref,
             bu_ref, wn_ref, bn_ref, o_ref):
        h = _layer_z(xh_ref, s_ref, ed_ref, a_ref, bm_ref, c_ref, d_ref,
                     cp_ref, bu_ref)
        xpn = jnp.dot(h, wn_ref[...], preferred_element_type=F32) + bn_ref[...]
        xb = xpn.astype(BF16)
        o_ref[0] = xb[:, :32]
        o_ref[1] = xb[:, 32:]

    wspec = lambda shape: pl.BlockSpec(shape, lambda i: (0, 0))
    return pl.pallas_call(
        body,
        grid=(N // _B,),
        in_specs=[
            pl.BlockSpec((2, _B, 32), lambda i: (0, i, 0)),
            pl.BlockSpec((2, _B, 32), lambda i: (0, i, 0)),
            pl.BlockSpec((2, _B, 32), lambda i: (0, i, 0)),
            wspec((64, 64)), wspec((64, 64)), wspec((16, 64)), wspec((64, 64)),
            wspec((1, 64)), wspec((1, 64)), wspec((64, 64)), wspec((1, 64)),
        ],
        out_specs=pl.BlockSpec((2, _B, 32), lambda i: (0, i, 0)),
        out_shape=jax.ShapeDtypeStruct((2, N, 32), BF16),
    )(xh, S, eadeg, A, Bm, C, D, cp, bu, Wn2, bn2.reshape(1, 64))


def _tc_fin(xh, S, eadeg, batch, A, Bm, C, D, cp, bu, G=64):
    N = xh.shape[1]
    nb = N // _B
    batch3 = batch.reshape(nb, 1, _B)

    def body(xh_ref, s_ref, ed_ref, b_ref, a_ref, bm_ref, c_ref, d_ref,
             cp_ref, bu_ref, o_ref, cnt_ref):
        i = pl.program_id(0)
        h = _layer_z(xh_ref, s_ref, ed_ref, a_ref, bm_ref, c_ref, d_ref,
                     cp_ref, bu_ref)
        b = b_ref[0, 0, :]
        iot = lax.broadcasted_iota(jnp.int32, (_B, G), 1)
        onehot = (b[:, None] == iot).astype(F32)

        @pl.when(i == 0)
        def _():
            o_ref[...] = jnp.zeros((G, 64), F32)
            cnt_ref[...] = jnp.zeros((8, G), F32)

        o_ref[...] += lax.dot_general(
            onehot, h, (((0,), (0,)), ((), ())), preferred_element_type=F32)
        cnt_ref[0, :] += jnp.sum(onehot, axis=0)

        @pl.when(i == nb - 1)
        def _():
            o_ref[...] = o_ref[...] / jnp.maximum(cnt_ref[0, :], 1.0)[:, None]

    wspec = lambda shape: pl.BlockSpec(shape, lambda i: (0, 0))
    return pl.pallas_call(
        body,
        grid=(nb,),
        in_specs=[
            pl.BlockSpec((2, _B, 32), lambda i: (0, i, 0)),
            pl.BlockSpec((2, _B, 32), lambda i: (0, i, 0)),
            pl.BlockSpec((2, _B, 32), lambda i: (0, i, 0)),
            pl.BlockSpec((1, 1, _B), lambda i: (i, 0, 0)),
            wspec((64, 64)), wspec((64, 64)), wspec((16, 64)), wspec((64, 64)),
            wspec((1, 64)), wspec((1, 64)),
        ],
        out_specs=pl.BlockSpec((G, 64), lambda i: (0, 0)),
        out_shape=jax.ShapeDtypeStruct((G, 64), F32),
        scratch_shapes=[pltpu.VMEM((8, G), F32)],
    )(xh, S, eadeg, batch3, A, Bm, C, D, cp, bu)


def _fuse(p):
    H = p['Wn'].shape[1]
    Wm1, Wm2, Wm3 = p['Wm'][:H], p['Wm'][H:2 * H], p['Wm'][2 * H:]
    Wu1, Wu2 = p['Wu'][:H], p['Wu'][H:]
    D = Wm3 @ Wu2
    A = Wu1 + D
    Bm = Wm1 @ Wu2
    cp = (p['bm'] + p['be'] @ Wm2) @ Wu2
    C = p['We'] @ Wm2 @ Wu2
    return A, Bm, C, D, cp.reshape(1, H), p['bu'].reshape(1, H)


def kernel(x, edge_index, edge_attr, batch, params):
    N = x.shape[0]
    E = edge_index.shape[1]
    EF = edge_attr.shape[1]
    src = edge_index[0]
    dst = edge_index[1]
    r0, _unused = _row_split(N)
    z32 = jnp.zeros((r0, 32), BF16)

    ea32 = _tc_ea32(edge_attr)
    eadeg = _sc_eadeg(N, E)(ea32, dst, z32)
    p1, p2, p3 = params['c1'], params['c2'], params['c3']
    xh = _tc_in(x, p1['Wn'], p1['bn'])
    seg = _sc_seg(N, E)
    out = None
    for li, p in enumerate((p1, p2, p3)):
        A, Bm, C, D, cp, bu = _fuse(p)
        S = seg(xh.reshape(2 * N, 32), src, dst, z32)
        if li < 2:
            pn = (p2, p3)[li]
            xh = _tc_mid(xh, S, eadeg, A, Bm, C, D, cp, bu, pn['Wn'], pn['bn'])
        else:
            out = _tc_fin(xh, S, eadeg, batch, A, Bm, C, D, cp, bu)
    return out


# speedup vs baseline: 1.1078x; 1.1078x over previous
import functools

import jax
import jax.numpy as jnp
from jax import lax
from jax.experimental import pallas as pl
from jax.experimental.pallas import tpu as pltpu
from jax.experimental.pallas import tpu_sc as plsc

F32 = jnp.float32
_NSUB = 16
_K = 128
_B = 2000


def _row_split(N):
    r0 = -(-((N + _NSUB - 1) // _NSUB) // 8) * 8
    last = N - (_NSUB - 1) * r0
    assert last > 0 and last % 8 == 0 and N % 8 == 0
    return r0, last


@functools.cache
def _sc_eadeg(N, E, EF):
    eps = E // _NSUB
    nfull = eps // _K
    tail = eps % _K
    r0, rlast = _row_split(N)
    mesh = plsc.VectorSubcoreMesh(core_axis_name="c", subcore_axis_name="s")

    def body(ea_hbm, dst_hbm, ones_hbm, z_hbm, out_hbm, acc, vbuf, dbuf, dtail):
        c = lax.axis_index("c")
        s = lax.axis_index("s")

        @pl.when(s < _NSUB - 1)
        def _():
            pltpu.sync_copy(z_hbm, acc.at[pl.ds(s * r0, r0)])

        @pl.when(s == _NSUB - 1)
        def _():
            pltpu.sync_copy(z_hbm.at[pl.ds(0, rlast)],
                            acc.at[pl.ds((_NSUB - 1) * r0, rlast)])
        pltpu.sync_copy(ones_hbm, vbuf)
        plsc.subcore_barrier()
        base = s * eps

        def step(j, carry):
            off = base + j * _K
            pltpu.sync_copy(dst_hbm.at[pl.ds(off, _K)], dbuf)

            @pl.when(c == 0)
            def _():
                pltpu.sync_copy(ea_hbm.at[pl.ds(off, _K)], vbuf)

            pltpu.sync_copy(vbuf, acc.at[dbuf], add=True)
            return carry

        lax.fori_loop(0, nfull, step, 0)
        if tail:
            off = base + nfull * _K
            pltpu.sync_copy(dst_hbm.at[pl.ds(off, tail)], dtail)

            @pl.when(c == 0)
            def _():
                pltpu.sync_copy(ea_hbm.at[pl.ds(off, tail)],
                                vbuf.at[pl.ds(0, tail)])

            pltpu.sync_copy(vbuf.at[pl.ds(0, tail)], acc.at[dtail], add=True)
        plsc.subcore_barrier()

        @pl.when(s < _NSUB - 1)
        def _():
            pltpu.sync_copy(acc.at[pl.ds(s * r0, r0)],
                            out_hbm.at[c, pl.ds(s * r0, r0)])

        @pl.when(s == _NSUB - 1)
        def _():
            pltpu.sync_copy(acc.at[pl.ds((_NSUB - 1) * r0, rlast)],
                            out_hbm.at[c, pl.ds((_NSUB - 1) * r0, rlast)])

    return pl.kernel(
        body,
        out_type=jax.ShapeDtypeStruct((2, N, EF), F32),
        mesh=mesh,
        compiler_params=pltpu.CompilerParams(use_tc_tiling_on_sc=False),
        scratch_types=[
            pltpu.VMEM_SHARED((N, EF), F32),
            pltpu.VMEM((_K, EF), F32),
            pltpu.VMEM((_K,), jnp.int32),
            pltpu.VMEM((max(tail, 16),), jnp.int32),
        ],
    )


BF16 = jnp.bfloat16


@functools.cache
def _sc_seg(N, E):
    eps = E // _NSUB
    nfull = eps // _K
    tail = eps % _K
    npairs = nfull // 2
    assert nfull == 2 * npairs
    r0, rlast = _row_split(N)
    mesh = plsc.VectorSubcoreMesh(core_axis_name="c", subcore_axis_name="s")

    def body(xh_hbm, src_hbm, dst_hbm, z_hbm, out_hbm, acc,
             gbufA, gbufB, sidxA, sidxB, gidxA, gidxB, didxA, didxB, dtail,
             semA, semB, semIA, semIB):
        c = lax.axis_index("c")
        s = lax.axis_index("s")
        base_row = c * N

        @pl.when(s < _NSUB - 1)
        def _():
            pltpu.sync_copy(z_hbm, acc.at[pl.ds(s * r0, r0)])

        @pl.when(s == _NSUB - 1)
        def _():
            pltpu.sync_copy(z_hbm.at[pl.ds(0, rlast)],
                            acc.at[pl.ds((_NSUB - 1) * r0, rlast)])

        plsc.subcore_barrier()
        base = s * eps

        def load_idx(j, sidx, didx, semI):
            off = base + j * _K
            pltpu.async_copy(src_hbm.at[pl.ds(off, _K)], sidx, semI)
            pltpu.async_copy(dst_hbm.at[pl.ds(off, _K)], didx, semI)

        def wait_idx(j, sidx, didx, semI):
            off = base + j * _K
            pltpu.make_async_copy(src_hbm.at[pl.ds(off, _K)], sidx, semI).wait()
            pltpu.make_async_copy(dst_hbm.at[pl.ds(off, _K)], didx, semI).wait()

        def compute_gidx(sidx, gidx, n16=_K // 16):
            for t in range(n16):
                sl = pl.ds(t * 16, 16)
                gidx[sl] = sidx[sl] + base_row

        def start_gather(gidx, gbuf, sem):
            pltpu.async_copy(xh_hbm.at[gidx], gbuf, sem)

        def wait_gather(gbuf, sem):
            pltpu.make_async_copy(xh_hbm.at[pl.ds(0, _K)], gbuf, sem).wait()

        def scatter(gbuf, didx):
            pltpu.sync_copy(gbuf, acc.at[didx], add=True)

        load_idx(0, sidxA, didxA, semIA)

        def pair(i, carry):
            j0 = 2 * i

            @pl.when(i > 0)
            def _():
                wait_gather(gbufB, semB)
                scatter(gbufB, didxB)

            load_idx(j0 + 1, sidxB, didxB, semIB)
            wait_idx(j0, sidxA, didxA, semIA)
            compute_gidx(sidxA, gidxA)
            start_gather(gidxA, gbufA, semA)
            wait_idx(j0 + 1, sidxB, didxB, semIB)
            compute_gidx(sidxB, gidxB)
            wait_gather(gbufA, semA)
            scatter(gbufA, didxA)
            start_gather(gidxB, gbufB, semB)

            @pl.when(i + 1 < npairs)
            def _():
                load_idx(j0 + 2, sidxA, didxA, semIA)

            return carry

        lax.fori_loop(0, npairs, pair, 0)
        wait_gather(gbufB, semB)
        scatter(gbufB, didxB)
        if tail:
            off = base + nfull * _K
            pltpu.sync_copy(src_hbm.at[pl.ds(off, tail)],
                            sidxA.at[pl.ds(0, tail)])
            pltpu.sync_copy(dst_hbm.at[pl.ds(off, tail)], dtail)
            compute_gidx(sidxA, gidxA, tail // 16)
            pltpu.async_copy(xh_hbm.at[gidxA.at[pl.ds(0, tail)]],
                             gbufA.at[pl.ds(0, tail)], semA).wait()
            pltpu.sync_copy(gbufA.at[pl.ds(0, tail)], acc.at[dtail], add=True)
        plsc.subcore_barrier()

        @pl.when(s < _NSUB - 1)
        def _():
            pltpu.sync_copy(acc.at[pl.ds(s * r0, r0)],
                            out_hbm.at[c, pl.ds(s * r0, r0)])

        @pl.when(s == _NSUB - 1)
        def _():
            pltpu.sync_copy(acc.at[pl.ds((_NSUB - 1) * r0, rlast)],
                            out_hbm.at[c, pl.ds((_NSUB - 1) * r0, rlast)])

    return pl.kernel(
        body,
        out_type=jax.ShapeDtypeStruct((2, N, 32), BF16),
        mesh=mesh,
        compiler_params=pltpu.CompilerParams(use_tc_tiling_on_sc=False),
        scratch_types=[
            pltpu.VMEM_SHARED((N, 32), BF16),
            pltpu.VMEM((_K, 32), BF16),
            pltpu.VMEM((_K, 32), BF16),
            pltpu.VMEM((_K,), jnp.int32),
            pltpu.VMEM((_K,), jnp.int32),
            pltpu.VMEM((_K,), jnp.int32),
            pltpu.VMEM((_K,), jnp.int32),
            pltpu.VMEM((_K,), jnp.int32),
            pltpu.VMEM((_K,), jnp.int32),
            pltpu.VMEM((max(tail, 16),), jnp.int32),
            pltpu.SemaphoreType.DMA,
            pltpu.SemaphoreType.DMA,
            pltpu.SemaphoreType.DMA,
            pltpu.SemaphoreType.DMA,
        ],
    )


def _tc_in(x, Wn, bn):
    N, Fin = x.shape

    def body(x_ref, w_ref, b_ref, o_ref):
        xp = jnp.dot(x_ref[...], w_ref[...], preferred_element_type=F32) \
            + b_ref[...]
        xb = xp.astype(BF16)
        o_ref[0] = xb[:, :32]
        o_ref[1] = xb[:, 32:]

    return pl.pallas_call(
        body,
        grid=(N // _B,),
        in_specs=[
            pl.BlockSpec((_B, Fin), lambda i: (i, 0)),
            pl.BlockSpec((Fin, 64), lambda i: (0, 0)),
            pl.BlockSpec((1, 64), lambda i: (0, 0)),
        ],
        out_specs=pl.BlockSpec((2, _B, 32), lambda i: (0, i, 0)),
        out_shape=jax.ShapeDtypeStruct((2, N, 32), BF16),
    )(x, Wn, bn.reshape(1, 64))


def _layer_z(xh_ref, s_ref, ed_ref, a_ref, bm_ref, c_ref, d_ref, cp_ref, bu_ref):
    xp = jnp.concatenate([xh_ref[0], xh_ref[1]], axis=1).astype(F32)
    sf = jnp.concatenate([s_ref[0], s_ref[1]], axis=1).astype(F32)
    ea = ed_ref[0]
    deg = ed_ref[1][:, 0:1] + 1.0
    z = (jnp.dot(xp, a_ref[...], preferred_element_type=F32)
         + (jnp.dot(xp, bm_ref[...], preferred_element_type=F32)
            + cp_ref[...]) * deg
         + jnp.dot(ea, c_ref[...], preferred_element_type=F32)
         + jnp.dot(sf, d_ref[...], preferred_element_type=F32)
         + bu_ref[...])
    return jnp.maximum(z, 0.0)


def _tc_mid(xh, S, eadeg, A, Bm, C, D, cp, bu, Wn2, bn2):
    N = xh.shape[1]

    def body(xh_ref, s_ref, ed_ref, a_ref, bm_ref, c_ref, d_ref, cp_ref,
             bu_ref, wn_ref, bn_ref, o_ref):
        h = _layer_z(xh_ref, s_ref, ed_ref, a_ref, bm_ref, c_ref, d_ref,
                     cp_ref, bu_ref)
        xpn = jnp.dot(h, wn_ref[...], preferred_element_type=F32) + bn_ref[...]
        xb = xpn.astype(BF16)
        o_ref[0] = xb[:, :32]
        o_ref[1] = xb[:, 32:]

    wspec = lambda shape: pl.BlockSpec(shape, lambda i: (0, 0))
    return pl.pallas_call(
        body,
        grid=(N // _B,),
        in_specs=[
            pl.BlockSpec((2, _B, 32), lambda i: (0, i, 0)),
            pl.BlockSpec((2, _B, 32), lambda i: (0, i, 0)),
            pl.BlockSpec((2, _B, 16), lambda i: (0, i, 0)),
            wspec((64, 64)), wspec((64, 64)), wspec((16, 64)), wspec((64, 64)),
            wspec((1, 64)), wspec((1, 64)), wspec((64, 64)), wspec((1, 64)),
        ],
        out_specs=pl.BlockSpec((2, _B, 32), lambda i: (0, i, 0)),
        out_shape=jax.ShapeDtypeStruct((2, N, 32), BF16),
    )(xh, S, eadeg, A, Bm, C, D, cp, bu, Wn2, bn2.reshape(1, 64))


def _tc_fin(xh, S, eadeg, batch, A, Bm, C, D, cp, bu, G=64):
    N = xh.shape[1]
    nb = N // _B
    batch3 = batch.reshape(nb, 1, _B)

    def body(xh_ref, s_ref, ed_ref, b_ref, a_ref, bm_ref, c_ref, d_ref,
             cp_ref, bu_ref, o_ref, cnt_ref):
        i = pl.program_id(0)
        h = _layer_z(xh_ref, s_ref, ed_ref, a_ref, bm_ref, c_ref, d_ref,
                     cp_ref, bu_ref)
        b = b_ref[0, 0, :]
        iot = lax.broadcasted_iota(jnp.int32, (_B, G), 1)
        onehot = (b[:, None] == iot).astype(F32)

        @pl.when(i == 0)
        def _():
            o_ref[...] = jnp.zeros((G, 64), F32)
            cnt_ref[...] = jnp.zeros((8, G), F32)

        o_ref[...] += lax.dot_general(
            onehot, h, (((0,), (0,)), ((), ())), preferred_element_type=F32)
        cnt_ref[0, :] += jnp.sum(onehot, axis=0)

        @pl.when(i == nb - 1)
        def _():
            o_ref[...] = o_ref[...] / jnp.maximum(cnt_ref[0, :], 1.0)[:, None]

    wspec = lambda shape: pl.BlockSpec(shape, lambda i: (0, 0))
    return pl.pallas_call(
        body,
        grid=(nb,),
        in_specs=[
            pl.BlockSpec((2, _B, 32), lambda i: (0, i, 0)),
            pl.BlockSpec((2, _B, 32), lambda i: (0, i, 0)),
            pl.BlockSpec((2, _B, 16), lambda i: (0, i, 0)),
            pl.BlockSpec((1, 1, _B), lambda i: (i, 0, 0)),
            wspec((64, 64)), wspec((64, 64)), wspec((16, 64)), wspec((64, 64)),
            wspec((1, 64)), wspec((1, 64)),
        ],
        out_specs=pl.BlockSpec((G, 64), lambda i: (0, 0)),
        out_shape=jax.ShapeDtypeStruct((G, 64), F32),
        scratch_shapes=[pltpu.VMEM((8, G), F32)],
    )(xh, S, eadeg, batch3, A, Bm, C, D, cp, bu)


def _fuse(p):
    H = p['Wn'].shape[1]
    Wm1, Wm2, Wm3 = p['Wm'][:H], p['Wm'][H:2 * H], p['Wm'][2 * H:]
    Wu1, Wu2 = p['Wu'][:H], p['Wu'][H:]
    D = Wm3 @ Wu2
    A = Wu1 + D
    Bm = Wm1 @ Wu2
    cp = (p['bm'] + p['be'] @ Wm2) @ Wu2
    C = p['We'] @ Wm2 @ Wu2
    return A, Bm, C, D, cp.reshape(1, H), p['bu'].reshape(1, H)


def kernel(x, edge_index, edge_attr, batch, params):
    N = x.shape[0]
    E = edge_index.shape[1]
    EF = edge_attr.shape[1]
    src = edge_index[0]
    dst = edge_index[1]
    r0, _unused = _row_split(N)
    ones16 = jnp.ones((_K, EF), F32)
    z16 = jnp.zeros((r0, EF), F32)
    z32 = jnp.zeros((r0, 32), BF16)

    eadeg = _sc_eadeg(N, E, EF)(edge_attr, dst, ones16, z16)
    p1, p2, p3 = params['c1'], params['c2'], params['c3']
    xh = _tc_in(x, p1['Wn'], p1['bn'])
    seg = _sc_seg(N, E)
    out = None
    for li, p in enumerate((p1, p2, p3)):
        A, Bm, C, D, cp, bu = _fuse(p)
        S = seg(xh.reshape(2 * N, 32), src, dst, z32)
        if li < 2:
            pn = (p2, p3)[li]
            xh = _tc_mid(xh, S, eadeg, A, Bm, C, D, cp, bu, pn['Wn'], pn['bn'])
        else:
            out = _tc_fin(xh, S, eadeg, batch, A, Bm, C, D, cp, bu)
    return out
